# Initial kernel scaffold; baseline (speedup 1.0000x reference)
import jax
import jax.numpy as jnp
from jax.experimental import pallas as pl


def kernel(x, adj, weight, bias):
    raise NotImplementedError("write your pallas kernel here")


# speedup vs baseline: 2.0273x; 2.0273x over previous
import functools

import jax
import jax.numpy as jnp
from jax.experimental import pallas as pl
from jax.experimental.pallas import tpu as pltpu

_SLOPE = 0.01


def _fused_gcn_kernel(adj_ref, x_ref, w_ref, b_ref, o_ref, *, nb, cin, cout):
    adj = adj_ref[...].astype(jnp.bfloat16)
    xs = x_ref[...].astype(jnp.bfloat16)
    y = jnp.dot(adj, xs, preferred_element_type=jnp.float32)
    w = w_ref[...].astype(jnp.bfloat16)
    b = b_ref[...]
    for k in range(nb):
        yk = y[:, k * cin:(k + 1) * cin].astype(jnp.bfloat16)
        h = jnp.dot(yk, w, preferred_element_type=jnp.float32) + b
        o_ref[:, k * cout:(k + 1) * cout] = jnp.where(h > 0, h, _SLOPE * h)


@jax.jit
def kernel(x, adj, weight, bias):
    M, B, Cin = x.shape
    Cout = weight.shape[1]

    x = x.astype(jnp.float32)
    adj = adj.astype(jnp.float32)
    weight = weight.astype(jnp.float32)
    bias2 = bias.astype(jnp.float32).reshape(1, Cout)

    nb = 16 if B % 16 == 0 else 1
    xf = x.reshape(M, B * Cin)

    out = pl.pallas_call(
        functools.partial(_fused_gcn_kernel, nb=nb, cin=Cin, cout=Cout),
        out_shape=jax.ShapeDtypeStruct((M, B * Cout), jnp.float32),
        grid=(B // nb,),
        in_specs=[
            pl.BlockSpec((M, M), lambda j: (0, 0)),
            pl.BlockSpec((M, nb * Cin), lambda j: (0, j)),
            pl.BlockSpec((Cin, Cout), lambda j: (0, 0)),
            pl.BlockSpec((1, Cout), lambda j: (0, 0)),
        ],
        out_specs=pl.BlockSpec((M, nb * Cout), lambda j: (0, j)),
        compiler_params=pltpu.CompilerParams(
            dimension_semantics=("parallel",)),
    )(adj, xf, weight, bias2)

    return out.reshape(M, B, Cout)


# speedup vs baseline: 5.3206x; 2.6244x over previous
import functools

import jax
import jax.numpy as jnp
from jax.experimental import pallas as pl
from jax.experimental.pallas import tpu as pltpu

_SLOPE = 0.01


def _fused_gcn_kernel(adj_ref, x_ref, w_ref, b_ref, o_ref, *, nb, cin, cout):
    M = adj_ref.shape[0]
    adj = adj_ref[...].astype(jnp.bfloat16)
    xs = x_ref[...].reshape(M, nb * cin).astype(jnp.bfloat16)
    y = jnp.dot(adj, xs, preferred_element_type=jnp.float32)
    w = w_ref[...].astype(jnp.bfloat16)
    b = b_ref[...]
    for k in range(nb):
        yk = y[:, k * cin:(k + 1) * cin].astype(jnp.bfloat16)
        h = jnp.dot(yk, w, preferred_element_type=jnp.float32) + b
        o_ref[:, k, :] = jnp.where(h > 0, h, _SLOPE * h)


@jax.jit
def kernel(x, adj, weight, bias):
    M, B, Cin = x.shape
    Cout = weight.shape[1]

    x = x.astype(jnp.float32)
    adj = adj.astype(jnp.float32)
    weight = weight.astype(jnp.float32)
    bias2 = bias.astype(jnp.float32).reshape(1, Cout)

    nb = 16 if B % 16 == 0 else 8

    out = pl.pallas_call(
        functools.partial(_fused_gcn_kernel, nb=nb, cin=Cin, cout=Cout),
        out_shape=jax.ShapeDtypeStruct((M, B, Cout), jnp.float32),
        grid=(B // nb,),
        in_specs=[
            pl.BlockSpec((M, M), lambda j: (0, 0)),
            pl.BlockSpec((M, nb, Cin), lambda j: (0, j, 0)),
            pl.BlockSpec((Cin, Cout), lambda j: (0, 0)),
            pl.BlockSpec((1, Cout), lambda j: (0, 0)),
        ],
        out_specs=pl.BlockSpec((M, nb, Cout), lambda j: (0, j, 0)),
        compiler_params=pltpu.CompilerParams(
            dimension_semantics=("parallel",)),
    )(adj, x, weight, bias2)

    return out


# speedup vs baseline: 7.1445x; 1.3428x over previous
import functools

import jax
import jax.numpy as jnp
from jax.experimental import pallas as pl
from jax.experimental.pallas import tpu as pltpu

_SLOPE = 0.01


def _fused_gcn_kernel(adj_ref, x_ref, w_ref, b_ref, o_ref, *, nb, cin, cout):
    M = adj_ref.shape[0]
    adj = adj_ref[...].astype(jnp.bfloat16)
    x2 = x_ref[...].astype(jnp.bfloat16).reshape(M, nb * cin)
    y = jnp.dot(adj, x2, preferred_element_type=jnp.float32)
    y2 = y.astype(jnp.bfloat16).reshape(M * nb, cin)
    w = w_ref[...].astype(jnp.bfloat16)
    h = jnp.dot(y2, w, preferred_element_type=jnp.float32) + b_ref[...]
    o_ref[...] = jnp.where(h > 0, h, _SLOPE * h).reshape(M, nb, cout)


@jax.jit
def kernel(x, adj, weight, bias):
    M, B, Cin = x.shape
    Cout = weight.shape[1]

    x = x.astype(jnp.float32)
    adj = adj.astype(jnp.float32)
    weight = weight.astype(jnp.float32)
    bias2 = bias.astype(jnp.float32).reshape(1, Cout)

    nb = 16 if B % 16 == 0 else 8

    out = pl.pallas_call(
        functools.partial(_fused_gcn_kernel, nb=nb, cin=Cin, cout=Cout),
        out_shape=jax.ShapeDtypeStruct((M, B, Cout), jnp.float32),
        grid=(B // nb,),
        in_specs=[
            pl.BlockSpec((M, M), lambda j: (0, 0)),
            pl.BlockSpec((M, nb, Cin), lambda j: (0, j, 0)),
            pl.BlockSpec((Cin, Cout), lambda j: (0, 0)),
            pl.BlockSpec((1, Cout), lambda j: (0, 0)),
        ],
        out_specs=pl.BlockSpec((M, nb, Cout), lambda j: (0, j, 0)),
        compiler_params=pltpu.CompilerParams(
            dimension_semantics=("parallel",)),
    )(adj, x, weight, bias2)

    return out


# speedup vs baseline: 7.1860x; 1.0058x over previous
import functools

import jax
import jax.numpy as jnp
from jax.experimental import pallas as pl
from jax.experimental.pallas import tpu as pltpu

_SLOPE = 0.01


def _fused_gcn_kernel(adj_ref, x_ref, w_ref, b_ref, o_ref, *, nb, cin, cout):
    M = adj_ref.shape[0]
    adj = adj_ref[...].astype(jnp.bfloat16)
    x2 = x_ref[...].astype(jnp.bfloat16).reshape(M, nb * cin)
    y = jnp.dot(adj, x2, preferred_element_type=jnp.float32)
    y2 = y.astype(jnp.bfloat16).reshape(M * nb, cin)
    w = w_ref[...].astype(jnp.bfloat16)
    h = jnp.dot(y2, w, preferred_element_type=jnp.float32) + b_ref[...]
    o_ref[...] = jnp.maximum(h, _SLOPE * h).reshape(M, nb, cout)


@jax.jit
def kernel(x, adj, weight, bias):
    M, B, Cin = x.shape
    Cout = weight.shape[1]

    x = x.astype(jnp.float32)
    adj = adj.astype(jnp.float32)
    weight = weight.astype(jnp.float32)
    bias2 = bias.astype(jnp.float32).reshape(1, Cout)

    nb = 16 if B % 16 == 0 else 8

    out = pl.pallas_call(
        functools.partial(_fused_gcn_kernel, nb=nb, cin=Cin, cout=Cout),
        out_shape=jax.ShapeDtypeStruct((M, B, Cout), jnp.float32),
        grid=(B // nb,),
        in_specs=[
            pl.BlockSpec((M, M), lambda j: (0, 0)),
            pl.BlockSpec((M, nb, Cin), lambda j: (0, j, 0)),
            pl.BlockSpec((Cin, Cout), lambda j: (0, 0)),
            pl.BlockSpec((1, Cout), lambda j: (0, 0)),
        ],
        out_specs=pl.BlockSpec((M, nb, Cout), lambda j: (0, j, 0)),
        compiler_params=pltpu.CompilerParams(
            dimension_semantics=("parallel",)),
    )(adj, x, weight, bias2)

    return out
